# baseline (speedup 1.0000x reference)
import functools

import jax
import jax.numpy as jnp
from jax import lax
from jax.experimental import pallas as pl
from jax.experimental.pallas import tpu as pltpu
from jax.experimental.pallas import tpu_sc as plsc

_NC = 1
_NS = 16
_NW = _NC * _NS

_K = 125
_NBUF = 1
_PASSES = 2


def _xw_body(f_ref, w_ref, o_ref):
    o_ref[0] = jnp.dot(f_ref[...], w_ref[0], preferred_element_type=jnp.float32)


def _combine_body(p_ref, f_ref, ws_ref, b_ref, o_ref):
    h = jnp.dot(f_ref[...], ws_ref[...], preferred_element_type=jnp.float32)
    o_ref[...] = h + jnp.sum(p_ref[...], axis=0) + b_ref[...]


def _make_sc_kernel(n_pad, d_out, chunks):
    mesh = plsc.VectorSubcoreMesh(
        core_axis_name="c", subcore_axis_name="s", num_cores=_NC, num_subcores=_NS
    )
    rows_per_tile = n_pad // _NS
    hc = chunks // _PASSES
    assert chunks % _PASSES == 0 and hc % _NBUF == 0
    assert _PASSES == 1 or hc % 8 == 0

    scratch = (
        [
            pltpu.VMEM((hc, _K), jnp.int32),
            pltpu.VMEM((hc, _K), jnp.int32),
            pltpu.VMEM_SHARED((n_pad, d_out), jnp.float32),
        ]
        + [pltpu.VMEM((_K, d_out), jnp.float32) for _ in range(_NBUF)]
        + [pltpu.SemaphoreType.DMA for _ in range(_NBUF)]
    )

    @functools.partial(
        pl.kernel,
        mesh=mesh,
        out_type=jax.ShapeDtypeStruct((_NC, n_pad, d_out), jnp.float32),
        scratch_types=scratch,
    )
    def sc_kernel(xw_hbm, gidx_hbm, dst_hbm, zero_hbm, out_hbm,
                  idx_v, dst_v, acc_sh, *bufs_and_sems):
        rows = bufs_and_sems[:_NBUF]
        sem_g = bufs_and_sems[_NBUF:2 * _NBUF]
        c = lax.axis_index("c")
        s = lax.axis_index("s")
        w = c * _NS + s

        r0 = s * rows_per_tile
        pltpu.sync_copy(
            zero_hbm.at[pl.ds(r0, rows_per_tile)],
            acc_sh.at[pl.ds(r0, rows_per_tile)],
        )
        plsc.subcore_barrier()

        def fire_gather(g, b):
            pltpu.async_copy(xw_hbm.at[idx_v.at[g]], rows[b], sem_g[b])

        def wait_gather(g, b):
            pltpu.make_async_copy(xw_hbm.at[idx_v.at[g]], rows[b], sem_g[b]).wait()

        def scatter_add(g, b):
            pltpu.sync_copy(rows[b], acc_sh.at[dst_v.at[g]], add=True)

        for p in range(_PASSES):
            pltpu.sync_copy(gidx_hbm.at[w, pl.ds(p * hc, hc)], idx_v)
            pltpu.sync_copy(dst_hbm.at[w, pl.ds(p * hc, hc)], dst_v)

            if _NBUF == 1:
                @pl.loop(0, hc)
                def _(g):
                    pltpu.async_copy(
                        xw_hbm.at[idx_v.at[g]], rows[0], sem_g[0]
                    ).wait()
                    scatter_add(g, 0)
            else:
                fire_gather(0, 0)

                @pl.loop(0, hc - _NBUF, step=_NBUF)
                def _(G):
                    for b in range(_NBUF):
                        g = G + b
                        wait_gather(g, b)
                        fire_gather(g + 1, (b + 1) % _NBUF)
                        scatter_add(g, b)

                for b in range(_NBUF):
                    g = hc - _NBUF + b
                    wait_gather(g, b)
                    if b + 1 < _NBUF:
                        fire_gather(g + 1, b + 1)
                    scatter_add(g, b)

        plsc.subcore_barrier()

        pltpu.sync_copy(
            acc_sh.at[pl.ds(r0, rows_per_tile)],
            out_hbm.at[c, pl.ds(r0, rows_per_tile)],
        )

    return sc_kernel


def kernel(features, edge_index, edge_type, W_rel, W_self, bias):
    n_nodes, d_in = features.shape
    n_rel, _, d_out = W_rel.shape
    n_edges = edge_type.shape[0]

    bn = 400
    n_blocks = n_nodes // bn

    xw = pl.pallas_call(
        _xw_body,
        grid=(n_rel, n_blocks),
        in_specs=[
            pl.BlockSpec((bn, d_in), lambda r, i: (i, 0)),
            pl.BlockSpec((1, d_in, d_out), lambda r, i: (r, 0, 0)),
        ],
        out_specs=pl.BlockSpec((1, bn, d_out), lambda r, i: (r, i, 0)),
        out_shape=jax.ShapeDtypeStruct((n_rel, n_nodes, d_out), jnp.float32),
    )(features, W_rel)
    xw_flat = xw.reshape(n_rel * n_nodes, d_out)

    src = edge_index[0]
    dst = edge_index[1]
    n_pad = ((n_nodes + 8 * _NS - 1) // (8 * _NS)) * (8 * _NS)
    per_tile = n_edges // _NW
    chunk_quant = _K * _NBUF * _PASSES * (8 if _PASSES > 1 else 1)
    per_tile_pad = ((per_tile + chunk_quant - 1) // chunk_quant) * chunk_quant
    chunks = per_tile_pad // _K
    pad = per_tile_pad - per_tile

    gidx = (edge_type * n_nodes + src).reshape(_NW, per_tile)
    dst2 = dst.reshape(_NW, per_tile)
    if pad:
        gidx = jnp.pad(gidx, ((0, 0), (0, pad)))
        n_scratch = n_pad - n_nodes
        pad_rows = n_nodes + (
            (jnp.arange(pad)[None, :] + 31 * jnp.arange(_NW)[:, None]) % n_scratch
        ).astype(jnp.int32)
        dst2 = jnp.concatenate([dst2, pad_rows], axis=1)
    gidx = gidx.reshape(_NW, chunks, _K)
    dst2 = dst2.reshape(_NW, chunks, _K)
    zeros_init = jnp.zeros((n_pad, d_out), jnp.float32)

    partials = _make_sc_kernel(n_pad, d_out, chunks)(
        xw_flat, gidx, dst2, zeros_init
    )

    h = pl.pallas_call(
        _combine_body,
        grid=(n_blocks,),
        in_specs=[
            pl.BlockSpec((_NC, bn, d_out), lambda i: (0, i, 0)),
            pl.BlockSpec((bn, d_in), lambda i: (i, 0)),
            pl.BlockSpec((d_in, d_out), lambda i: (0, 0)),
            pl.BlockSpec((d_out,), lambda i: (0,)),
        ],
        out_specs=pl.BlockSpec((bn, d_out), lambda i: (i, 0)),
        out_shape=jax.ShapeDtypeStruct((n_nodes, d_out), jnp.float32),
    )(partials, features, W_self, bias)
    return h


# speedup vs baseline: 1.3174x; 1.3174x over previous
import functools

import jax
import jax.numpy as jnp
from jax import lax
from jax.experimental import pallas as pl
from jax.experimental.pallas import tpu as pltpu
from jax.experimental.pallas import tpu_sc as plsc

_NC = 2
_NS = 16
_NW = _NC * _NS

_K = 80
_NBUF = 1
_PASSES = 1


def _xw_body(f_ref, w_ref, o_ref):
    o_ref[0] = jnp.dot(f_ref[...], w_ref[0], preferred_element_type=jnp.float32)


def _combine_body(p_ref, f_ref, ws_ref, b_ref, o_ref):
    h = jnp.dot(f_ref[...], ws_ref[...], preferred_element_type=jnp.float32)
    o_ref[...] = h + jnp.sum(p_ref[...], axis=0) + b_ref[...]


def _make_sc_kernel(n_pad, d_out, chunks):
    mesh = plsc.VectorSubcoreMesh(
        core_axis_name="c", subcore_axis_name="s", num_cores=_NC, num_subcores=_NS
    )
    rows_per_tile = n_pad // _NS
    hc = chunks // _PASSES
    assert chunks % _PASSES == 0 and hc % _NBUF == 0
    assert _PASSES == 1 or hc % 8 == 0

    scratch = (
        [
            pltpu.VMEM((hc, _K), jnp.int32),
            pltpu.VMEM((hc, _K), jnp.int32),
            pltpu.VMEM_SHARED((n_pad, d_out), jnp.float32),
        ]
        + [pltpu.VMEM((_K, d_out), jnp.float32) for _ in range(_NBUF)]
        + [pltpu.SemaphoreType.DMA for _ in range(_NBUF)]
    )

    @functools.partial(
        pl.kernel,
        mesh=mesh,
        out_type=jax.ShapeDtypeStruct((_NC, n_pad, d_out), jnp.float32),
        scratch_types=scratch,
    )
    def sc_kernel(xw_hbm, gidx_hbm, dst_hbm, zero_hbm, out_hbm,
                  idx_v, dst_v, acc_sh, *bufs_and_sems):
        rows = bufs_and_sems[:_NBUF]
        sem_g = bufs_and_sems[_NBUF:2 * _NBUF]
        c = lax.axis_index("c")
        s = lax.axis_index("s")
        w = c * _NS + s

        r0 = s * rows_per_tile
        pltpu.sync_copy(
            zero_hbm.at[pl.ds(r0, rows_per_tile)],
            acc_sh.at[pl.ds(r0, rows_per_tile)],
        )
        plsc.subcore_barrier()

        def fire_gather(g, b):
            pltpu.async_copy(xw_hbm.at[idx_v.at[g]], rows[b], sem_g[b])

        def wait_gather(g, b):
            pltpu.make_async_copy(xw_hbm.at[idx_v.at[g]], rows[b], sem_g[b]).wait()

        def scatter_add(g, b):
            pltpu.sync_copy(rows[b], acc_sh.at[dst_v.at[g]], add=True)

        for p in range(_PASSES):
            pltpu.sync_copy(gidx_hbm.at[w, pl.ds(p * hc, hc)], idx_v)
            pltpu.sync_copy(dst_hbm.at[w, pl.ds(p * hc, hc)], dst_v)

            if _NBUF == 1:
                @pl.loop(0, hc)
                def _(g):
                    pltpu.async_copy(
                        xw_hbm.at[idx_v.at[g]], rows[0], sem_g[0]
                    ).wait()
                    scatter_add(g, 0)
            else:
                fire_gather(0, 0)

                @pl.loop(0, hc - _NBUF, step=_NBUF)
                def _(G):
                    for b in range(_NBUF):
                        g = G + b
                        wait_gather(g, b)
                        fire_gather(g + 1, (b + 1) % _NBUF)
                        scatter_add(g, b)

                for b in range(_NBUF):
                    g = hc - _NBUF + b
                    wait_gather(g, b)
                    if b + 1 < _NBUF:
                        fire_gather(g + 1, b + 1)
                    scatter_add(g, b)

        plsc.subcore_barrier()

        pltpu.sync_copy(
            acc_sh.at[pl.ds(r0, rows_per_tile)],
            out_hbm.at[c, pl.ds(r0, rows_per_tile)],
        )

    return sc_kernel


def kernel(features, edge_index, edge_type, W_rel, W_self, bias):
    n_nodes, d_in = features.shape
    n_rel, _, d_out = W_rel.shape
    n_edges = edge_type.shape[0]

    bn = 400
    n_blocks = n_nodes // bn

    xw = pl.pallas_call(
        _xw_body,
        grid=(n_rel, n_blocks),
        in_specs=[
            pl.BlockSpec((bn, d_in), lambda r, i: (i, 0)),
            pl.BlockSpec((1, d_in, d_out), lambda r, i: (r, 0, 0)),
        ],
        out_specs=pl.BlockSpec((1, bn, d_out), lambda r, i: (r, i, 0)),
        out_shape=jax.ShapeDtypeStruct((n_rel, n_nodes, d_out), jnp.float32),
    )(features, W_rel)
    xw_flat = xw.reshape(n_rel * n_nodes, d_out)

    src = edge_index[0]
    dst = edge_index[1]
    n_pad = ((n_nodes + 8 * _NS - 1) // (8 * _NS)) * (8 * _NS)
    per_tile = n_edges // _NW
    chunk_quant = _K * _NBUF * _PASSES * (8 if _PASSES > 1 else 1)
    per_tile_pad = ((per_tile + chunk_quant - 1) // chunk_quant) * chunk_quant
    chunks = per_tile_pad // _K
    pad = per_tile_pad - per_tile

    gidx = (edge_type * n_nodes + src).reshape(_NW, per_tile)
    dst2 = dst.reshape(_NW, per_tile)
    if pad:
        gidx = jnp.pad(gidx, ((0, 0), (0, pad)))
        n_scratch = n_pad - n_nodes
        pad_rows = n_nodes + (
            (jnp.arange(pad)[None, :] + 31 * jnp.arange(_NW)[:, None]) % n_scratch
        ).astype(jnp.int32)
        dst2 = jnp.concatenate([dst2, pad_rows], axis=1)
    gidx = gidx.reshape(_NW, chunks, _K)
    dst2 = dst2.reshape(_NW, chunks, _K)
    zeros_init = jnp.zeros((n_pad, d_out), jnp.float32)

    partials = _make_sc_kernel(n_pad, d_out, chunks)(
        xw_flat, gidx, dst2, zeros_init
    )

    h = pl.pallas_call(
        _combine_body,
        grid=(n_blocks,),
        in_specs=[
            pl.BlockSpec((_NC, bn, d_out), lambda i: (0, i, 0)),
            pl.BlockSpec((bn, d_in), lambda i: (i, 0)),
            pl.BlockSpec((d_in, d_out), lambda i: (0, 0)),
            pl.BlockSpec((d_out,), lambda i: (0,)),
        ],
        out_specs=pl.BlockSpec((bn, d_out), lambda i: (i, 0)),
        out_shape=jax.ShapeDtypeStruct((n_nodes, d_out), jnp.float32),
    )(partials, features, W_self, bias)
    return h


# speedup vs baseline: 1.7882x; 1.3573x over previous
import functools

import jax
import jax.numpy as jnp
from jax import lax
from jax.experimental import pallas as pl
from jax.experimental.pallas import tpu as pltpu
from jax.experimental.pallas import tpu_sc as plsc

_NC = 2
_NS = 16
_NW = _NC * _NS

_K = 80
_NBUF = 1
_PASSES = 1


def _xw_body(f_ref, w_ref, o_ref):
    o_ref[0] = jnp.dot(f_ref[...], w_ref[0], preferred_element_type=jnp.float32)


def _combine_body(p_ref, f_ref, ws_ref, b_ref, o_ref):
    h = jnp.dot(f_ref[...], ws_ref[...], preferred_element_type=jnp.float32)
    o_ref[...] = h + jnp.sum(p_ref[...], axis=0) + b_ref[...]


def _make_sc_kernel(n_pad, d_out, chunks):
    mesh = plsc.VectorSubcoreMesh(
        core_axis_name="c", subcore_axis_name="s", num_cores=_NC, num_subcores=_NS
    )
    rows_per_tile = n_pad // _NS
    hc = chunks // _PASSES
    assert chunks % _PASSES == 0 and hc % _NBUF == 0
    assert _PASSES == 1 or hc % 8 == 0

    scratch = (
        [
            pltpu.VMEM((hc, _K), jnp.int32),
            pltpu.VMEM((hc, _K), jnp.int32),
            pltpu.VMEM_SHARED((n_pad, d_out), jnp.float32),
        ]
        + [pltpu.VMEM((_K, d_out), jnp.float32) for _ in range(_NBUF)]
        + [pltpu.SemaphoreType.DMA for _ in range(_NBUF)]
    )

    @functools.partial(
        pl.kernel,
        mesh=mesh,
        out_type=jax.ShapeDtypeStruct((_NC, n_pad, d_out), jnp.float32),
        scratch_types=scratch,
    )
    def sc_kernel(xw_hbm, gidx_hbm, dst_hbm, zero_hbm, out_hbm,
                  idx_v, dst_v, acc_sh, *bufs_and_sems):
        rows = bufs_and_sems[:_NBUF]
        sem_g = bufs_and_sems[_NBUF:2 * _NBUF]
        c = lax.axis_index("c")
        s = lax.axis_index("s")
        w = c * _NS + s

        r0 = s * rows_per_tile
        pltpu.sync_copy(
            zero_hbm.at[pl.ds(r0, rows_per_tile)],
            acc_sh.at[pl.ds(r0, rows_per_tile)],
        )
        plsc.subcore_barrier()

        def fire_gather(g, b):
            pltpu.async_copy(xw_hbm.at[idx_v.at[g]], rows[b], sem_g[b])

        def wait_gather(g, b):
            pltpu.make_async_copy(xw_hbm.at[idx_v.at[g]], rows[b], sem_g[b]).wait()

        def scatter_add(g, b):
            pltpu.sync_copy(rows[b], acc_sh.at[dst_v.at[g]], add=True)

        for p in range(_PASSES):
            pltpu.sync_copy(gidx_hbm.at[w, pl.ds(p * hc, hc)], idx_v)
            pltpu.sync_copy(dst_hbm.at[w, pl.ds(p * hc, hc)], dst_v)

            if _NBUF == 1:
                @pl.loop(0, hc)
                def _(g):
                    pltpu.async_copy(
                        xw_hbm.at[idx_v.at[g]], rows[0], sem_g[0]
                    ).wait()
                    scatter_add(g, 0)
            else:
                fire_gather(0, 0)

                @pl.loop(0, hc - _NBUF, step=_NBUF)
                def _(G):
                    for b in range(_NBUF):
                        g = G + b
                        wait_gather(g, b)
                        fire_gather(g + 1, (b + 1) % _NBUF)
                        scatter_add(g, b)

                for b in range(_NBUF):
                    g = hc - _NBUF + b
                    wait_gather(g, b)
                    if b + 1 < _NBUF:
                        fire_gather(g + 1, b + 1)
                    scatter_add(g, b)

        plsc.subcore_barrier()

        pltpu.sync_copy(
            acc_sh.at[pl.ds(r0, rows_per_tile)],
            out_hbm.at[c, pl.ds(r0, rows_per_tile)],
        )

    return sc_kernel


def kernel(features, edge_index, edge_type, W_rel, W_self, bias):
    n_nodes, d_in = features.shape
    n_rel, _, d_out = W_rel.shape
    n_edges = edge_type.shape[0]

    bn = 2000
    n_blocks = n_nodes // bn

    xw = pl.pallas_call(
        _xw_body,
        grid=(n_blocks, n_rel),
        in_specs=[
            pl.BlockSpec((bn, d_in), lambda i, r: (i, 0)),
            pl.BlockSpec((1, d_in, d_out), lambda i, r: (r, 0, 0)),
        ],
        out_specs=pl.BlockSpec((1, bn, d_out), lambda i, r: (r, i, 0)),
        out_shape=jax.ShapeDtypeStruct((n_rel, n_nodes, d_out), jnp.float32),
    )(features, W_rel)
    xw_flat = xw.reshape(n_rel * n_nodes, d_out)

    src = edge_index[0]
    dst = edge_index[1]
    n_pad = ((n_nodes + 8 * _NS - 1) // (8 * _NS)) * (8 * _NS)
    per_tile = n_edges // _NW
    chunk_quant = _K * _NBUF * _PASSES * (8 if _PASSES > 1 else 1)
    per_tile_pad = ((per_tile + chunk_quant - 1) // chunk_quant) * chunk_quant
    chunks = per_tile_pad // _K
    pad = per_tile_pad - per_tile

    gidx = (edge_type * n_nodes + src).reshape(_NW, per_tile)
    dst2 = dst.reshape(_NW, per_tile)
    if pad:
        gidx = jnp.pad(gidx, ((0, 0), (0, pad)))
        n_scratch = n_pad - n_nodes
        pad_rows = n_nodes + (
            (jnp.arange(pad)[None, :] + 31 * jnp.arange(_NW)[:, None]) % n_scratch
        ).astype(jnp.int32)
        dst2 = jnp.concatenate([dst2, pad_rows], axis=1)
    gidx = gidx.reshape(_NW, chunks, _K)
    dst2 = dst2.reshape(_NW, chunks, _K)
    zeros_init = jnp.zeros((n_pad, d_out), jnp.float32)

    partials = _make_sc_kernel(n_pad, d_out, chunks)(
        xw_flat, gidx, dst2, zeros_init
    )

    h = pl.pallas_call(
        _combine_body,
        grid=(n_blocks,),
        in_specs=[
            pl.BlockSpec((_NC, bn, d_out), lambda i: (0, i, 0)),
            pl.BlockSpec((bn, d_in), lambda i: (i, 0)),
            pl.BlockSpec((d_in, d_out), lambda i: (0, 0)),
            pl.BlockSpec((d_out,), lambda i: (0,)),
        ],
        out_specs=pl.BlockSpec((bn, d_out), lambda i: (i, 0)),
        out_shape=jax.ShapeDtypeStruct((n_nodes, d_out), jnp.float32),
    )(partials, features, W_self, bias)
    return h


# speedup vs baseline: 1.9075x; 1.0667x over previous
import functools

import jax
import jax.numpy as jnp
from jax import lax
from jax.experimental import pallas as pl
from jax.experimental.pallas import tpu as pltpu
from jax.experimental.pallas import tpu_sc as plsc

_NC = 2
_NS = 16
_NW = _NC * _NS

_K = 100
_NBUF = 1
_PASSES = 1


def _xw_body(f_ref, w_ref, o_ref):
    o_ref[0] = jnp.dot(f_ref[...], w_ref[0], preferred_element_type=jnp.float32)


def _combine_body(p_ref, f_ref, ws_ref, b_ref, o_ref):
    h = jnp.dot(f_ref[...], ws_ref[...], preferred_element_type=jnp.float32)
    o_ref[...] = h + jnp.sum(p_ref[...], axis=0) + b_ref[...]


def _make_sc_kernel(n_pad, d_out, chunks):
    mesh = plsc.VectorSubcoreMesh(
        core_axis_name="c", subcore_axis_name="s", num_cores=_NC, num_subcores=_NS
    )
    rows_per_tile = n_pad // _NS
    hc = chunks // _PASSES
    assert chunks % _PASSES == 0 and hc % _NBUF == 0
    assert _PASSES == 1 or hc % 8 == 0

    scratch = (
        [
            pltpu.VMEM((hc, _K), jnp.int32),
            pltpu.VMEM((hc, _K), jnp.int32),
            pltpu.VMEM_SHARED((n_pad, d_out), jnp.float32),
        ]
        + [pltpu.VMEM((_K, d_out), jnp.float32) for _ in range(_NBUF)]
        + [pltpu.SemaphoreType.DMA for _ in range(_NBUF)]
    )

    @functools.partial(
        pl.kernel,
        mesh=mesh,
        out_type=jax.ShapeDtypeStruct((_NC, n_pad, d_out), jnp.float32),
        scratch_types=scratch,
    )
    def sc_kernel(xw_hbm, gidx_hbm, dst_hbm, zero_hbm, out_hbm,
                  idx_v, dst_v, acc_sh, *bufs_and_sems):
        rows = bufs_and_sems[:_NBUF]
        sem_g = bufs_and_sems[_NBUF:2 * _NBUF]
        c = lax.axis_index("c")
        s = lax.axis_index("s")
        w = c * _NS + s

        r0 = s * rows_per_tile
        pltpu.sync_copy(
            zero_hbm.at[pl.ds(r0, rows_per_tile)],
            acc_sh.at[pl.ds(r0, rows_per_tile)],
        )
        plsc.subcore_barrier()

        def fire_gather(g, b):
            pltpu.async_copy(xw_hbm.at[idx_v.at[g]], rows[b], sem_g[b])

        def wait_gather(g, b):
            pltpu.make_async_copy(xw_hbm.at[idx_v.at[g]], rows[b], sem_g[b]).wait()

        def scatter_add(g, b):
            pltpu.sync_copy(rows[b], acc_sh.at[dst_v.at[g]], add=True)

        for p in range(_PASSES):
            pltpu.sync_copy(gidx_hbm.at[w, pl.ds(p * hc, hc)], idx_v)
            pltpu.sync_copy(dst_hbm.at[w, pl.ds(p * hc, hc)], dst_v)

            if _NBUF == 1:
                @pl.loop(0, hc)
                def _(g):
                    pltpu.async_copy(
                        xw_hbm.at[idx_v.at[g]], rows[0], sem_g[0]
                    ).wait()
                    scatter_add(g, 0)
            else:
                fire_gather(0, 0)

                @pl.loop(0, hc - _NBUF, step=_NBUF)
                def _(G):
                    for b in range(_NBUF):
                        g = G + b
                        wait_gather(g, b)
                        fire_gather(g + 1, (b + 1) % _NBUF)
                        scatter_add(g, b)

                for b in range(_NBUF):
                    g = hc - _NBUF + b
                    wait_gather(g, b)
                    if b + 1 < _NBUF:
                        fire_gather(g + 1, b + 1)
                    scatter_add(g, b)

        plsc.subcore_barrier()

        pltpu.sync_copy(
            acc_sh.at[pl.ds(r0, rows_per_tile)],
            out_hbm.at[c, pl.ds(r0, rows_per_tile)],
        )

    return sc_kernel


def kernel(features, edge_index, edge_type, W_rel, W_self, bias):
    n_nodes, d_in = features.shape
    n_rel, _, d_out = W_rel.shape
    n_edges = edge_type.shape[0]

    bn = 2000
    n_blocks = n_nodes // bn

    xw = pl.pallas_call(
        _xw_body,
        grid=(n_blocks, n_rel),
        in_specs=[
            pl.BlockSpec((bn, d_in), lambda i, r: (i, 0)),
            pl.BlockSpec((1, d_in, d_out), lambda i, r: (r, 0, 0)),
        ],
        out_specs=pl.BlockSpec((1, bn, d_out), lambda i, r: (r, i, 0)),
        out_shape=jax.ShapeDtypeStruct((n_rel, n_nodes, d_out), jnp.float32),
    )(features, W_rel)
    xw_flat = xw.reshape(n_rel * n_nodes, d_out)

    src = edge_index[0]
    dst = edge_index[1]
    n_pad = ((n_nodes + 8 * _NS - 1) // (8 * _NS)) * (8 * _NS)
    per_tile = n_edges // _NW
    chunk_quant = _K * _NBUF * _PASSES * (8 if _PASSES > 1 else 1)
    per_tile_pad = ((per_tile + chunk_quant - 1) // chunk_quant) * chunk_quant
    chunks = per_tile_pad // _K
    pad = per_tile_pad - per_tile

    gidx = (edge_type * n_nodes + src).reshape(_NW, per_tile)
    dst2 = dst.reshape(_NW, per_tile)
    if pad:
        gidx = jnp.pad(gidx, ((0, 0), (0, pad)))
        n_scratch = n_pad - n_nodes
        pad_rows = n_nodes + (
            (jnp.arange(pad)[None, :] + 31 * jnp.arange(_NW)[:, None]) % n_scratch
        ).astype(jnp.int32)
        dst2 = jnp.concatenate([dst2, pad_rows], axis=1)
    gidx = gidx.reshape(_NW, chunks, _K)
    dst2 = dst2.reshape(_NW, chunks, _K)
    zeros_init = jnp.zeros((n_pad, d_out), jnp.float32)

    partials = _make_sc_kernel(n_pad, d_out, chunks)(
        xw_flat, gidx, dst2, zeros_init
    )

    h = pl.pallas_call(
        _combine_body,
        grid=(n_blocks,),
        in_specs=[
            pl.BlockSpec((_NC, bn, d_out), lambda i: (0, i, 0)),
            pl.BlockSpec((bn, d_in), lambda i: (i, 0)),
            pl.BlockSpec((d_in, d_out), lambda i: (0, 0)),
            pl.BlockSpec((d_out,), lambda i: (0,)),
        ],
        out_specs=pl.BlockSpec((bn, d_out), lambda i: (i, 0)),
        out_shape=jax.ShapeDtypeStruct((n_nodes, d_out), jnp.float32),
    )(partials, features, W_self, bias)
    return h


# speedup vs baseline: 1.9994x; 1.0482x over previous
import functools

import jax
import jax.numpy as jnp
from jax import lax
from jax.experimental import pallas as pl
from jax.experimental.pallas import tpu as pltpu
from jax.experimental.pallas import tpu_sc as plsc

_NC = 2
_NS = 16
_NW = _NC * _NS

_K = 125
_NBUF = 1
_PASSES = 1


def _xw_body(f_ref, w_ref, o_ref):
    o_ref[0] = jnp.dot(f_ref[...], w_ref[0], preferred_element_type=jnp.float32)


def _combine_body(p_ref, f_ref, ws_ref, b_ref, o_ref):
    h = jnp.dot(f_ref[...], ws_ref[...], preferred_element_type=jnp.float32)
    o_ref[...] = h + jnp.sum(p_ref[...], axis=0) + b_ref[...]


def _make_sc_kernel(n_pad, d_out, chunks):
    mesh = plsc.VectorSubcoreMesh(
        core_axis_name="c", subcore_axis_name="s", num_cores=_NC, num_subcores=_NS
    )
    rows_per_tile = n_pad // _NS
    hc = chunks // _PASSES
    assert chunks % _PASSES == 0 and hc % _NBUF == 0
    assert _PASSES == 1 or hc % 8 == 0

    scratch = (
        [
            pltpu.VMEM((hc, _K), jnp.int32),
            pltpu.VMEM((hc, _K), jnp.int32),
            pltpu.VMEM_SHARED((n_pad, d_out), jnp.float32),
        ]
        + [pltpu.VMEM((_K, d_out), jnp.float32) for _ in range(_NBUF)]
        + [pltpu.SemaphoreType.DMA for _ in range(_NBUF)]
    )

    @functools.partial(
        pl.kernel,
        mesh=mesh,
        out_type=jax.ShapeDtypeStruct((_NC, n_pad, d_out), jnp.float32),
        scratch_types=scratch,
    )
    def sc_kernel(xw_hbm, gidx_hbm, dst_hbm, zero_hbm, out_hbm,
                  idx_v, dst_v, acc_sh, *bufs_and_sems):
        rows = bufs_and_sems[:_NBUF]
        sem_g = bufs_and_sems[_NBUF:2 * _NBUF]
        c = lax.axis_index("c")
        s = lax.axis_index("s")
        w = c * _NS + s

        r0 = s * rows_per_tile
        pltpu.sync_copy(
            zero_hbm.at[pl.ds(r0, rows_per_tile)],
            acc_sh.at[pl.ds(r0, rows_per_tile)],
        )
        plsc.subcore_barrier()

        def fire_gather(g, b):
            pltpu.async_copy(xw_hbm.at[idx_v.at[g]], rows[b], sem_g[b])

        def wait_gather(g, b):
            pltpu.make_async_copy(xw_hbm.at[idx_v.at[g]], rows[b], sem_g[b]).wait()

        def scatter_add(g, b):
            pltpu.sync_copy(rows[b], acc_sh.at[dst_v.at[g]], add=True)

        for p in range(_PASSES):
            pltpu.sync_copy(gidx_hbm.at[w, pl.ds(p * hc, hc)], idx_v)
            pltpu.sync_copy(dst_hbm.at[w, pl.ds(p * hc, hc)], dst_v)

            if _NBUF == 1:
                @pl.loop(0, hc)
                def _(g):
                    pltpu.async_copy(
                        xw_hbm.at[idx_v.at[g]], rows[0], sem_g[0]
                    ).wait()
                    scatter_add(g, 0)
            else:
                fire_gather(0, 0)

                @pl.loop(0, hc - _NBUF, step=_NBUF)
                def _(G):
                    for b in range(_NBUF):
                        g = G + b
                        wait_gather(g, b)
                        fire_gather(g + 1, (b + 1) % _NBUF)
                        scatter_add(g, b)

                for b in range(_NBUF):
                    g = hc - _NBUF + b
                    wait_gather(g, b)
                    if b + 1 < _NBUF:
                        fire_gather(g + 1, b + 1)
                    scatter_add(g, b)

        plsc.subcore_barrier()

        pltpu.sync_copy(
            acc_sh.at[pl.ds(r0, rows_per_tile)],
            out_hbm.at[c, pl.ds(r0, rows_per_tile)],
        )

    return sc_kernel


def kernel(features, edge_index, edge_type, W_rel, W_self, bias):
    n_nodes, d_in = features.shape
    n_rel, _, d_out = W_rel.shape
    n_edges = edge_type.shape[0]

    bn = 2000
    n_blocks = n_nodes // bn

    xw = pl.pallas_call(
        _xw_body,
        grid=(n_blocks, n_rel),
        in_specs=[
            pl.BlockSpec((bn, d_in), lambda i, r: (i, 0)),
            pl.BlockSpec((1, d_in, d_out), lambda i, r: (r, 0, 0)),
        ],
        out_specs=pl.BlockSpec((1, bn, d_out), lambda i, r: (r, i, 0)),
        out_shape=jax.ShapeDtypeStruct((n_rel, n_nodes, d_out), jnp.float32),
    )(features, W_rel)
    xw_flat = xw.reshape(n_rel * n_nodes, d_out)

    src = edge_index[0]
    dst = edge_index[1]
    n_pad = ((n_nodes + 8 * _NS - 1) // (8 * _NS)) * (8 * _NS)
    per_tile = n_edges // _NW
    chunk_quant = _K * _NBUF * _PASSES * (8 if _PASSES > 1 else 1)
    per_tile_pad = ((per_tile + chunk_quant - 1) // chunk_quant) * chunk_quant
    chunks = per_tile_pad // _K
    pad = per_tile_pad - per_tile

    gidx = (edge_type * n_nodes + src).reshape(_NW, per_tile)
    dst2 = dst.reshape(_NW, per_tile)
    if pad:
        gidx = jnp.pad(gidx, ((0, 0), (0, pad)))
        n_scratch = n_pad - n_nodes
        pad_rows = n_nodes + (
            (jnp.arange(pad)[None, :] + 31 * jnp.arange(_NW)[:, None]) % n_scratch
        ).astype(jnp.int32)
        dst2 = jnp.concatenate([dst2, pad_rows], axis=1)
    gidx = gidx.reshape(_NW, chunks, _K)
    dst2 = dst2.reshape(_NW, chunks, _K)
    zeros_init = jnp.zeros((n_pad, d_out), jnp.float32)

    partials = _make_sc_kernel(n_pad, d_out, chunks)(
        xw_flat, gidx, dst2, zeros_init
    )

    h = pl.pallas_call(
        _combine_body,
        grid=(n_blocks,),
        in_specs=[
            pl.BlockSpec((_NC, bn, d_out), lambda i: (0, i, 0)),
            pl.BlockSpec((bn, d_in), lambda i: (i, 0)),
            pl.BlockSpec((d_in, d_out), lambda i: (0, 0)),
            pl.BlockSpec((d_out,), lambda i: (0,)),
        ],
        out_specs=pl.BlockSpec((bn, d_out), lambda i: (i, 0)),
        out_shape=jax.ShapeDtypeStruct((n_nodes, d_out), jnp.float32),
    )(partials, features, W_self, bias)
    return h
